# baseline (speedup 1.0000x reference)
import functools

import jax
import jax.numpy as jnp
from jax import lax
from jax.experimental import pallas as pl
from jax.experimental.pallas import tpu as pltpu
from jax.experimental.pallas import tpu_sc as plsc

B = 64
T = 2048
D = 256
H = 64
L16 = 16
NW = 32
RPT = B // NW


def _usum_body(x_ref, wu_ref, ws2_ref, us_ref, e_ref, acc_ref):
    i = pl.program_id(0)
    j = pl.program_id(1)
    ni = pl.num_programs(0)
    nj = pl.num_programs(1)
    bb = us_ref.shape[0]

    @pl.when(j == 0)
    def _():
        us_ref[...] = jnp.zeros_like(us_ref)

    us_ref[...] += jnp.sum(x_ref[...], axis=1)

    @pl.when(j == nj - 1)
    def _():
        acc_ref[pl.ds(i * bb, bb), :] = us_ref[...]

    @pl.when((i == ni - 1) & (j == nj - 1))
    def _():
        v2 = jax.lax.dot_general(ws2_ref[...], wu_ref[...],
                                 (((1,), (0,)), ((), ())),
                                 preferred_element_type=jnp.float32)
        su = jax.lax.dot_general(v2, acc_ref[...],
                                 (((1,), (1,)), ((), ())),
                                 preferred_element_type=jnp.float32)
        su = su * (1.0 / T)
        e_ref[...] = jnp.exp(su - jnp.max(su, axis=1, keepdims=True))


def _u_sum_and_e(latent_ys, Wu, ws2, bb=8, tb=2048):
    return pl.pallas_call(
        _usum_body,
        grid=(B // bb, T // tb),
        in_specs=[
            pl.BlockSpec((bb, tb, D), lambda i, j: (i, j, 0)),
            pl.BlockSpec((H, D), lambda i, j: (0, 0)),
            pl.BlockSpec((1, H), lambda i, j: (0, 0)),
        ],
        out_specs=[
            pl.BlockSpec((bb, D), lambda i, j: (i, 0)),
            pl.BlockSpec((1, B), lambda i, j: (0, 0)),
        ],
        out_shape=[
            jax.ShapeDtypeStruct((B, D), jnp.float32),
            jax.ShapeDtypeStruct((1, B), jnp.float32),
        ],
        scratch_shapes=[pltpu.VMEM((B, D), jnp.float32)],
    )(latent_ys, Wu, ws2)


def _sc_partition(e):
    mesh = plsc.VectorSubcoreMesh(core_axis_name="c", subcore_axis_name="s")

    @functools.partial(
        pl.kernel, mesh=mesh,
        out_type=jax.ShapeDtypeStruct((B, 3 * B), jnp.float32),
        scratch_types=[
            pltpu.VMEM((B,), jnp.float32),
            pltpu.VMEM((3 * B,), jnp.float32),
            pltpu.VMEM((3 * L16,), jnp.float32),
        ],
    )
    def k(e_hbm, out_hbm, e_v, mrow_v, red_v):
        vf = lambda s: jnp.full((L16,), s, jnp.float32)
        vi = lambda s: jnp.full((L16,), s, jnp.int32)
        wid = lax.axis_index("s") * 2 + lax.axis_index("c")
        pltpu.sync_copy(e_hbm, e_v)
        lane = lax.iota(jnp.int32, L16)
        nch = B // L16

        def allred(x, op):
            acc = x
            for s in (8, 4, 2, 1):
                red_v[pl.ds(L16, L16)] = acc
                up = red_v[pl.ds(L16 + s, L16)]
                dn = red_v[pl.ds(L16 - s, L16)]
                sel = (lane & vi(s)) == vi(0)
                acc = op(acc, jnp.where(sel, up, dn))
            return acc

        zero = vf(0.0)
        one = vf(1.0)
        evs = [e_v[pl.ds(L16 * c, L16)] for c in range(nch)]
        jvs = [lane + vi(L16 * c) for c in range(nch)]
        jfs = [jv.astype(jnp.float32) for jv in jvs]
        for r in range(RPT):
            i = wid * RPT + r
            iv = vi(i)
            act = [jnp.where(jv != iv, one, zero) for jv in jvs]
            cnt = vf(float(B - 1))
            ssum4 = zero
            for a, ev in zip(act, evs):
                ssum4 = ssum4 + a * ev
            ssum = allred(ssum4, jnp.add)
            for l in range(2):
                mean = ssum / jnp.maximum(cnt, one)
                low = [a * jnp.where(ev <= mean, one, zero)
                       for a, ev in zip(act, evs)]
                lcnt4 = zero
                lsum4 = zero
                for lo, ev in zip(low, evs):
                    lcnt4 = lcnt4 + lo
                    lsum4 = lsum4 + lo * ev
                lcnt = allred(lcnt4, jnp.add)
                lsum = allred(lsum4, jnp.add)
                inv_v = jnp.where(lcnt > zero,
                                  one / jnp.maximum(lcnt, one), zero)
                for c in range(nch):
                    mrow_v[pl.ds(l * B + L16 * c, L16)] = low[c] * inv_v
                act = [a * (one - lo) for a, lo in zip(act, low)]
                cnt = cnt - lcnt
                ssum = ssum - lsum
            acnt = cnt
            ainv_v = one / jnp.maximum(acnt, one)
            em = [jnp.where(jv != iv, ev, -one) for jv, ev in zip(jvs, evs)]
            m4 = jnp.maximum(jnp.maximum(em[0], em[1]),
                             jnp.maximum(em[2], em[3]))
            rmax = allred(m4, jnp.maximum)
            bigf = vf(float(B))
            cand4 = bigf
            for x, jf in zip(em, jfs):
                cand4 = jnp.minimum(cand4, jnp.where(x == rmax, jf, bigf))
            jstar = allred(cand4, jnp.minimum)
            fbm = jnp.where(acnt > zero, zero, one)
            for c in range(nch):
                onehot = jnp.where(jfs[c] == jstar, one, zero)
                mrow_v[pl.ds(2 * B + L16 * c, L16)] = (
                    fbm * onehot + (one - fbm) * act[c] * ainv_v)
            pltpu.sync_copy(mrow_v, out_hbm.at[i])

    return k(e)


def _fuse_body(us_ref, m_ref, lw_ref, wf2_ref, bf_ref, x_ref, w_ref, g_ref,
               b_ref, out_ref, cc_ref):
    i = pl.program_id(0)

    @pl.when(i == 0)
    def _():
        u_repr = us_ref[...] * (1.0 / T)
        m = m_ref[...]
        cross = (lw_ref[0] * jnp.dot(m[:, :B], u_repr,
                                     preferred_element_type=jnp.float32)
                 + lw_ref[1] * jnp.dot(m[:, B:2 * B], u_repr,
                                       preferred_element_type=jnp.float32)
                 + lw_ref[2] * jnp.dot(m[:, 2 * B:], u_repr,
                                       preferred_element_type=jnp.float32))
        cc_ref[...] = jax.lax.dot_general(
            cross, wf2_ref[...], (((1,), (1,)), ((), ())),
            preferred_element_type=jnp.float32) + bf_ref[...]

    x = x_ref[0]
    h = jax.lax.dot_general(x, w_ref[...], (((1,), (1,)), ((), ())),
                            preferred_element_type=jnp.float32)
    h = h + x + cc_ref[pl.ds(i, 1), :]
    mu = jnp.mean(h, axis=-1, keepdims=True)
    d = h - mu
    var = jnp.mean(d * d, axis=-1, keepdims=True)
    out_ref[0] = d * jax.lax.rsqrt(var + 1e-5) * g_ref[...] + b_ref[...]


def _fuse(u_sum, M, level_weights, Wf2, bf, latent_ys, Wf1, gamma, beta):
    cst = lambda shape: pl.BlockSpec(shape, lambda i: tuple(0 for _ in shape))
    return pl.pallas_call(
        _fuse_body,
        grid=(B,),
        in_specs=[
            cst((B, D)),
            cst((B, 3 * B)),
            pl.BlockSpec(memory_space=pltpu.SMEM),
            cst((D, D)),
            cst((1, D)),
            pl.BlockSpec((1, T, D), lambda i: (i, 0, 0)),
            cst((D, D)),
            cst((1, D)),
            cst((1, D)),
        ],
        out_specs=pl.BlockSpec((1, T, D), lambda i: (i, 0, 0)),
        out_shape=jax.ShapeDtypeStruct((B, T, D), jnp.float32),
        scratch_shapes=[pltpu.VMEM((B, D), jnp.float32)],
    )(u_sum, M, level_weights, Wf2, bf.reshape(1, D), latent_ys, Wf1,
      gamma.reshape(1, D), beta.reshape(1, D))


@jax.jit
def kernel(latent_ys, Wx, bx, Wu, bu, Ws, bs, level_weights, Wf, bf, gamma,
           beta):
    ws2 = Ws[:, H:]
    u_sum, e = _u_sum_and_e(latent_ys, Wu, ws2)
    M = _sc_partition(e.reshape(B))
    return _fuse(u_sum, M, level_weights, Wf[:, D:], bf, latent_ys,
                 Wf[:, :D], gamma, beta)


# speedup vs baseline: 1.1361x; 1.1361x over previous
import functools

import jax
import jax.numpy as jnp
from jax import lax
from jax.experimental import pallas as pl
from jax.experimental.pallas import tpu as pltpu
from jax.experimental.pallas import tpu_sc as plsc

B = 64
T = 2048
D = 256
H = 64
L16 = 16
NW = 32
RPT = B // NW


def _usum_body(x_ref, wu_ref, ws2_ref, us_ref, e_ref, acc_ref):
    i = pl.program_id(0)
    j = pl.program_id(1)
    ni = pl.num_programs(0)
    nj = pl.num_programs(1)
    bb = us_ref.shape[0]

    @pl.when(j == 0)
    def _():
        us_ref[...] = jnp.zeros_like(us_ref)

    us_ref[...] += jnp.sum(x_ref[...], axis=1)

    @pl.when(j == nj - 1)
    def _():
        acc_ref[pl.ds(i * bb, bb), :] = us_ref[...]

    @pl.when((i == ni - 1) & (j == nj - 1))
    def _():
        v2 = jax.lax.dot_general(ws2_ref[...], wu_ref[...],
                                 (((1,), (0,)), ((), ())),
                                 preferred_element_type=jnp.float32)
        su = jax.lax.dot_general(v2, acc_ref[...],
                                 (((1,), (1,)), ((), ())),
                                 preferred_element_type=jnp.float32)
        su = su * (1.0 / T)
        e_ref[...] = jnp.exp(su - jnp.max(su, axis=1, keepdims=True))


def _u_sum_and_e(latent_ys, Wu, ws2, bb=8, tb=1024):
    return pl.pallas_call(
        _usum_body,
        grid=(B // bb, T // tb),
        in_specs=[
            pl.BlockSpec((bb, tb, D), lambda i, j: (i, j, 0)),
            pl.BlockSpec((H, D), lambda i, j: (0, 0)),
            pl.BlockSpec((1, H), lambda i, j: (0, 0)),
        ],
        out_specs=[
            pl.BlockSpec((bb, D), lambda i, j: (i, 0)),
            pl.BlockSpec((1, B), lambda i, j: (0, 0)),
        ],
        out_shape=[
            jax.ShapeDtypeStruct((B, D), jnp.float32),
            jax.ShapeDtypeStruct((1, B), jnp.float32),
        ],
        scratch_shapes=[pltpu.VMEM((B, D), jnp.float32)],
    )(latent_ys, Wu, ws2)


def _sc_partition(e):
    mesh = plsc.VectorSubcoreMesh(core_axis_name="c", subcore_axis_name="s")

    @functools.partial(
        pl.kernel, mesh=mesh,
        out_type=jax.ShapeDtypeStruct((B, 3 * B), jnp.float32),
        scratch_types=[
            pltpu.VMEM((B,), jnp.float32),
            pltpu.VMEM((3 * B,), jnp.float32),
            pltpu.VMEM((3 * L16,), jnp.float32),
        ],
    )
    def k(e_hbm, out_hbm, e_v, mrow_v, red_v):
        vf = lambda s: jnp.full((L16,), s, jnp.float32)
        vi = lambda s: jnp.full((L16,), s, jnp.int32)
        wid = lax.axis_index("s") * 2 + lax.axis_index("c")
        pltpu.sync_copy(e_hbm, e_v)
        lane = lax.iota(jnp.int32, L16)
        nch = B // L16

        def allred(x, op):
            acc = x
            for s in (8, 4, 2, 1):
                red_v[pl.ds(L16, L16)] = acc
                up = red_v[pl.ds(L16 + s, L16)]
                dn = red_v[pl.ds(L16 - s, L16)]
                sel = (lane & vi(s)) == vi(0)
                acc = op(acc, jnp.where(sel, up, dn))
            return acc

        zero = vf(0.0)
        one = vf(1.0)
        evs = [e_v[pl.ds(L16 * c, L16)] for c in range(nch)]
        jvs = [lane + vi(L16 * c) for c in range(nch)]
        jfs = [jv.astype(jnp.float32) for jv in jvs]
        for r in range(RPT):
            i = wid * RPT + r
            iv = vi(i)
            act = [jnp.where(jv != iv, one, zero) for jv in jvs]
            cnt = vf(float(B - 1))
            ssum4 = zero
            for a, ev in zip(act, evs):
                ssum4 = ssum4 + a * ev
            ssum = allred(ssum4, jnp.add)
            for l in range(2):
                mean = ssum / jnp.maximum(cnt, one)
                low = [a * jnp.where(ev <= mean, one, zero)
                       for a, ev in zip(act, evs)]
                lcnt4 = zero
                lsum4 = zero
                for lo, ev in zip(low, evs):
                    lcnt4 = lcnt4 + lo
                    lsum4 = lsum4 + lo * ev
                lcnt = allred(lcnt4, jnp.add)
                lsum = allred(lsum4, jnp.add)
                inv_v = jnp.where(lcnt > zero,
                                  one / jnp.maximum(lcnt, one), zero)
                for c in range(nch):
                    mrow_v[pl.ds(l * B + L16 * c, L16)] = low[c] * inv_v
                act = [a * (one - lo) for a, lo in zip(act, low)]
                cnt = cnt - lcnt
                ssum = ssum - lsum
            acnt = cnt
            ainv_v = one / jnp.maximum(acnt, one)
            em = [jnp.where(jv != iv, ev, -one) for jv, ev in zip(jvs, evs)]
            m4 = jnp.maximum(jnp.maximum(em[0], em[1]),
                             jnp.maximum(em[2], em[3]))
            rmax = allred(m4, jnp.maximum)
            bigf = vf(float(B))
            cand4 = bigf
            for x, jf in zip(em, jfs):
                cand4 = jnp.minimum(cand4, jnp.where(x == rmax, jf, bigf))
            jstar = allred(cand4, jnp.minimum)
            fbm = jnp.where(acnt > zero, zero, one)
            for c in range(nch):
                onehot = jnp.where(jfs[c] == jstar, one, zero)
                mrow_v[pl.ds(2 * B + L16 * c, L16)] = (
                    fbm * onehot + (one - fbm) * act[c] * ainv_v)
            pltpu.sync_copy(mrow_v, out_hbm.at[i])

    return k(e)


def _fuse_body(us_ref, m_ref, lw_ref, wf2_ref, bf_ref, x_ref, w_ref, g_ref,
               b_ref, out_ref, cc_ref):
    i = pl.program_id(0)

    @pl.when(i == 0)
    def _():
        u_repr = us_ref[...] * (1.0 / T)
        m = m_ref[...]
        cross = (lw_ref[0] * jnp.dot(m[:, :B], u_repr,
                                     preferred_element_type=jnp.float32)
                 + lw_ref[1] * jnp.dot(m[:, B:2 * B], u_repr,
                                       preferred_element_type=jnp.float32)
                 + lw_ref[2] * jnp.dot(m[:, 2 * B:], u_repr,
                                       preferred_element_type=jnp.float32))
        cc_ref[...] = jax.lax.dot_general(
            cross, wf2_ref[...], (((1,), (1,)), ((), ())),
            preferred_element_type=jnp.float32) + bf_ref[...]

    nb = x_ref.shape[0]
    for s in range(nb):
        x = x_ref[s]
        h = jax.lax.dot_general(x, w_ref[...], (((1,), (1,)), ((), ())),
                                preferred_element_type=jnp.float32)
        h = h + x + cc_ref[pl.ds(i * nb + s, 1), :]
        mu = jnp.mean(h, axis=-1, keepdims=True)
        d = h - mu
        var = jnp.mean(d * d, axis=-1, keepdims=True)
        out_ref[s] = d * jax.lax.rsqrt(var + 1e-5) * g_ref[...] + b_ref[...]


def _fuse(u_sum, M, level_weights, Wf2, bf, latent_ys, Wf1, gamma, beta):
    nb = 2
    cst = lambda shape: pl.BlockSpec(shape, lambda i: tuple(0 for _ in shape))
    return pl.pallas_call(
        _fuse_body,
        grid=(B // nb,),
        in_specs=[
            cst((B, D)),
            cst((B, 3 * B)),
            pl.BlockSpec(memory_space=pltpu.SMEM),
            cst((D, D)),
            cst((1, D)),
            pl.BlockSpec((2, T, D), lambda i: (i, 0, 0)),
            cst((D, D)),
            cst((1, D)),
            cst((1, D)),
        ],
        out_specs=pl.BlockSpec((2, T, D), lambda i: (i, 0, 0)),
        out_shape=jax.ShapeDtypeStruct((B, T, D), jnp.float32),
        scratch_shapes=[pltpu.VMEM((B, D), jnp.float32)],
    )(u_sum, M, level_weights, Wf2, bf.reshape(1, D), latent_ys, Wf1,
      gamma.reshape(1, D), beta.reshape(1, D))


@jax.jit
def kernel(latent_ys, Wx, bx, Wu, bu, Ws, bs, level_weights, Wf, bf, gamma,
           beta):
    ws2 = Ws[:, H:]
    u_sum, e = _u_sum_and_e(latent_ys, Wu, ws2)
    M = _sc_partition(e.reshape(B))
    return _fuse(u_sum, M, level_weights, Wf[:, D:], bf, latent_ys,
                 Wf[:, :D], gamma, beta)


# speedup vs baseline: 1.1857x; 1.0436x over previous
import functools

import jax
import jax.numpy as jnp
from jax import lax
from jax.experimental import pallas as pl
from jax.experimental.pallas import tpu as pltpu
from jax.experimental.pallas import tpu_sc as plsc

B = 64
T = 2048
D = 256
H = 64
L16 = 16
NW = 32
RPT = B // NW


def _usum_body(x_ref, wu_ref, ws2_ref, us_ref, e_ref, acc_ref):
    i = pl.program_id(0)
    j = pl.program_id(1)
    ni = pl.num_programs(0)
    nj = pl.num_programs(1)
    bb = us_ref.shape[0]

    @pl.when(j == 0)
    def _():
        us_ref[...] = jnp.zeros_like(us_ref)

    us_ref[...] += jnp.sum(x_ref[...], axis=1)

    @pl.when(j == nj - 1)
    def _():
        acc_ref[pl.ds(i * bb, bb), :] = us_ref[...]

    @pl.when((i == ni - 1) & (j == nj - 1))
    def _():
        v2 = jax.lax.dot_general(ws2_ref[...], wu_ref[...],
                                 (((1,), (0,)), ((), ())),
                                 preferred_element_type=jnp.float32)
        su = jax.lax.dot_general(v2, acc_ref[...],
                                 (((1,), (1,)), ((), ())),
                                 preferred_element_type=jnp.float32)
        su = su * (1.0 / T)
        e_ref[...] = jnp.exp(su - jnp.max(su, axis=1, keepdims=True))


def _u_sum_and_e(latent_ys, Wu, ws2, bb=8, tb=1024):
    return pl.pallas_call(
        _usum_body,
        grid=(B // bb, T // tb),
        in_specs=[
            pl.BlockSpec((bb, tb, D), lambda i, j: (i, j, 0)),
            pl.BlockSpec((H, D), lambda i, j: (0, 0)),
            pl.BlockSpec((1, H), lambda i, j: (0, 0)),
        ],
        out_specs=[
            pl.BlockSpec((bb, D), lambda i, j: (i, 0)),
            pl.BlockSpec((1, B), lambda i, j: (0, 0)),
        ],
        out_shape=[
            jax.ShapeDtypeStruct((B, D), jnp.float32),
            jax.ShapeDtypeStruct((1, B), jnp.float32),
        ],
        scratch_shapes=[pltpu.VMEM((B, D), jnp.float32)],
    )(latent_ys, Wu, ws2)


def _sc_partition(e):
    mesh = plsc.VectorSubcoreMesh(core_axis_name="c", subcore_axis_name="s")

    @functools.partial(
        pl.kernel, mesh=mesh,
        out_type=jax.ShapeDtypeStruct((B, 3 * B), jnp.float32),
        scratch_types=[
            pltpu.VMEM((B,), jnp.float32),
            pltpu.VMEM((3 * B,), jnp.float32),
            pltpu.VMEM((3 * L16,), jnp.float32),
        ],
    )
    def k(e_hbm, out_hbm, e_v, mrow_v, red_v):
        vf = lambda s: jnp.full((L16,), s, jnp.float32)
        vi = lambda s: jnp.full((L16,), s, jnp.int32)
        wid = lax.axis_index("s") * 2 + lax.axis_index("c")
        pltpu.sync_copy(e_hbm, e_v)
        lane = lax.iota(jnp.int32, L16)
        nch = B // L16

        def allred(x, op):
            acc = x
            for s in (8, 4, 2, 1):
                red_v[pl.ds(L16, L16)] = acc
                up = red_v[pl.ds(L16 + s, L16)]
                dn = red_v[pl.ds(L16 - s, L16)]
                sel = (lane & vi(s)) == vi(0)
                acc = op(acc, jnp.where(sel, up, dn))
            return acc

        zero = vf(0.0)
        one = vf(1.0)
        evs = [e_v[pl.ds(L16 * c, L16)] for c in range(nch)]
        jvs = [lane + vi(L16 * c) for c in range(nch)]
        jfs = [jv.astype(jnp.float32) for jv in jvs]
        for r in range(RPT):
            i = wid * RPT + r
            iv = vi(i)
            act = [jnp.where(jv != iv, one, zero) for jv in jvs]
            cnt = vf(float(B - 1))
            ssum4 = zero
            for a, ev in zip(act, evs):
                ssum4 = ssum4 + a * ev
            ssum = allred(ssum4, jnp.add)
            for l in range(2):
                mean = ssum / jnp.maximum(cnt, one)
                low = [a * jnp.where(ev <= mean, one, zero)
                       for a, ev in zip(act, evs)]
                lcnt4 = zero
                lsum4 = zero
                for lo, ev in zip(low, evs):
                    lcnt4 = lcnt4 + lo
                    lsum4 = lsum4 + lo * ev
                lcnt = allred(lcnt4, jnp.add)
                lsum = allred(lsum4, jnp.add)
                inv_v = jnp.where(lcnt > zero,
                                  one / jnp.maximum(lcnt, one), zero)
                for c in range(nch):
                    mrow_v[pl.ds(l * B + L16 * c, L16)] = low[c] * inv_v
                act = [a * (one - lo) for a, lo in zip(act, low)]
                cnt = cnt - lcnt
                ssum = ssum - lsum
            acnt = cnt
            ainv_v = one / jnp.maximum(acnt, one)
            em = [jnp.where(jv != iv, ev, -one) for jv, ev in zip(jvs, evs)]
            m4 = jnp.maximum(jnp.maximum(em[0], em[1]),
                             jnp.maximum(em[2], em[3]))
            rmax = allred(m4, jnp.maximum)
            bigf = vf(float(B))
            cand4 = bigf
            for x, jf in zip(em, jfs):
                cand4 = jnp.minimum(cand4, jnp.where(x == rmax, jf, bigf))
            jstar = allred(cand4, jnp.minimum)
            fbm = jnp.where(acnt > zero, zero, one)
            for c in range(nch):
                onehot = jnp.where(jfs[c] == jstar, one, zero)
                mrow_v[pl.ds(2 * B + L16 * c, L16)] = (
                    fbm * onehot + (one - fbm) * act[c] * ainv_v)
            pltpu.sync_copy(mrow_v, out_hbm.at[i])

    return k(e)


def _fuse_body(us_ref, m_ref, lw_ref, wf2_ref, bf_ref, x_ref, w_ref, g_ref,
               b_ref, out_ref, cc_ref):
    i = pl.program_id(0)

    @pl.when(i == 0)
    def _():
        u_repr = us_ref[...] * (1.0 / T)
        m = m_ref[...]
        cross = (lw_ref[0] * jnp.dot(m[:, :B], u_repr,
                                     preferred_element_type=jnp.float32)
                 + lw_ref[1] * jnp.dot(m[:, B:2 * B], u_repr,
                                       preferred_element_type=jnp.float32)
                 + lw_ref[2] * jnp.dot(m[:, 2 * B:], u_repr,
                                       preferred_element_type=jnp.float32))
        cc_ref[...] = jax.lax.dot_general(
            cross, wf2_ref[...], (((1,), (1,)), ((), ())),
            preferred_element_type=jnp.float32) + bf_ref[...]

    nb = x_ref.shape[0]
    for s in range(nb):
        x = x_ref[s]
        h = jax.lax.dot_general(x, w_ref[...], (((1,), (1,)), ((), ())),
                                preferred_element_type=jnp.float32)
        h = h + x + cc_ref[pl.ds(i * nb + s, 1), :]
        mu = jnp.mean(h, axis=-1, keepdims=True)
        d = h - mu
        var = jnp.mean(d * d, axis=-1, keepdims=True)
        out_ref[s] = d * jax.lax.rsqrt(var + 1e-5) * g_ref[...] + b_ref[...]


def _fuse(u_sum, M, level_weights, Wf2, bf, latent_ys, Wf1, gamma, beta):
    nb = 4
    cst = lambda shape: pl.BlockSpec(shape, lambda i: tuple(0 for _ in shape))
    return pl.pallas_call(
        _fuse_body,
        grid=(B // nb,),
        in_specs=[
            cst((B, D)),
            cst((B, 3 * B)),
            pl.BlockSpec(memory_space=pltpu.SMEM),
            cst((D, D)),
            cst((1, D)),
            pl.BlockSpec((4, T, D), lambda i: (i, 0, 0)),
            cst((D, D)),
            cst((1, D)),
            cst((1, D)),
        ],
        out_specs=pl.BlockSpec((4, T, D), lambda i: (i, 0, 0)),
        out_shape=jax.ShapeDtypeStruct((B, T, D), jnp.float32),
        scratch_shapes=[pltpu.VMEM((B, D), jnp.float32)],
    )(u_sum, M, level_weights, Wf2, bf.reshape(1, D), latent_ys, Wf1,
      gamma.reshape(1, D), beta.reshape(1, D))


@jax.jit
def kernel(latent_ys, Wx, bx, Wu, bu, Ws, bs, level_weights, Wf, bf, gamma,
           beta):
    ws2 = Ws[:, H:]
    u_sum, e = _u_sum_and_e(latent_ys, Wu, ws2)
    M = _sc_partition(e.reshape(B))
    return _fuse(u_sum, M, level_weights, Wf[:, D:], bf, latent_ys,
                 Wf[:, :D], gamma, beta)
